# Initial kernel scaffold; baseline (speedup 1.0000x reference)
import jax
import jax.numpy as jnp
from jax.experimental import pallas as pl


def kernel(emb1, emb2, W1, b1, W2, b2):
    raise NotImplementedError("write your pallas kernel here")


# speedup vs baseline: 4.8192x; 4.8192x over previous
import functools

import jax
import jax.numpy as jnp
from jax import lax
from jax.experimental import pallas as pl
from jax.experimental.pallas import tpu as pltpu
from jax.experimental.pallas import tpu_sc as plsc

ALPHA = 3.0
K = 20

_NUM_WORKERS = 32
_LANES = 128


def _pick_block_rows(n):
    for br in (200, 128, 100, 64, 40, 16, 8):
        if n % br == 0 and br % 8 == 0:
            return br
    return n


def _nodevec_body(e1, w1, b1, e2, w2, b2, n1, n2):
    n1[...] = jnp.tanh(
        ALPHA * (lax.dot_general(e1[...], w1[...], (((1,), (1,)), ((), ())),
                                 preferred_element_type=jnp.float32) + b1[...]))
    n2[...] = jnp.tanh(
        ALPHA * (lax.dot_general(e2[...], w2[...], (((1,), (1,)), ((), ())),
                                 preferred_element_type=jnp.float32) + b2[...]))


def _nodevecs(emb1, emb2, W1, b1, W2, b2):
    n, d = emb1.shape
    out = jax.ShapeDtypeStruct((n, d), jnp.float32)
    return pl.pallas_call(
        _nodevec_body,
        out_shape=(out, out),
    )(emb1, W1, b1.reshape(1, d), emb2, W2, b2.reshape(1, d))


def _select_body(n1_ref, n2_ref, n1t_ref, n2t_ref, noise_ref,
                 pos_ref, valc_ref, valr_ref, off_smem, *, n, br, trash):
    b = pl.program_id(0)

    @pl.when(b == 0)
    def _():
        off_smem[0] = 0

    base = off_smem[0]
    r0 = b * br
    n1b = n1_ref[pl.ds(r0, br), :]
    n2b = n2_ref[pl.ds(r0, br), :]
    m1 = lax.dot_general(n1b, n2t_ref[...], (((1,), (0,)), ((), ())),
                         preferred_element_type=jnp.float32)
    m2 = lax.dot_general(n2b, n1t_ref[...], (((1,), (0,)), ((), ())),
                         preferred_element_type=jnp.float32)
    adj = jnp.maximum(jnp.tanh(ALPHA * (m1 - m2)), 0.0)
    v = adj + noise_ref[...]

    rowmax = jnp.max(v, axis=1, keepdims=True)
    hi = lax.bitcast_convert_type(rowmax, jnp.int32) + 1
    lo = jnp.zeros_like(hi)

    def bs_body(_, carry):
        blo, bhi = carry
        mid = (blo + bhi) // 2
        t = lax.bitcast_convert_type(mid, jnp.float32)
        cnt = jnp.sum((v >= t).astype(jnp.float32), axis=1, keepdims=True)
        ge = cnt >= float(K)
        return jnp.where(ge, mid, blo), jnp.where(ge, bhi, mid)

    lo, hi = lax.fori_loop(0, 31, bs_body, (lo, hi))
    t20 = lax.bitcast_convert_type(lo, jnp.float32)

    n_gt = jnp.sum((v > t20).astype(jnp.float32), axis=1, keepdims=True)
    r_need = float(K) - n_gt
    eq = v == t20
    colid = lax.broadcasted_iota(jnp.int32, (br, n), 1)
    rn = jnp.maximum(r_need, 1.0)
    clo = jnp.full((br, 1), -1, jnp.int32)
    chi = jnp.full((br, 1), n - 1, jnp.int32)

    def ts_body(_, carry):
        tlo, thi = carry
        mid = (tlo + thi) // 2
        cm = jnp.sum(jnp.where(eq & (colid <= mid), 1.0, 0.0), axis=1,
                     keepdims=True)
        ge = cm >= rn
        return jnp.where(ge, tlo, mid), jnp.where(ge, mid, thi)

    _, cstar = lax.fori_loop(0, 15, ts_body, (clo, chi))
    sel = (v > t20) | (eq & (colid <= cstar) & (r_need >= 1.0))
    kept = sel & (adj > 0.0)

    counts = jnp.sum(kept.astype(jnp.float32), axis=1, keepdims=True)
    tri = (lax.broadcasted_iota(jnp.int32, (br, br), 0)
           > lax.broadcasted_iota(jnp.int32, (br, br), 1)).astype(jnp.float32)
    offs = lax.dot_general(tri, counts, (((1,), (0,)), ((), ())),
                           preferred_element_type=jnp.float32)
    offs_i = base + offs.astype(jnp.int32)

    big = jnp.int32(n)
    rem = kept
    cols_list = []
    for _ in range(K):
        cj = jnp.min(jnp.where(rem, colid, big), axis=1, keepdims=True)
        cols_list.append(cj)
        rem = rem & (colid != cj)
    cols = jnp.concatenate(cols_list, axis=1)

    jidx = lax.broadcasted_iota(jnp.int32, (br, K), 1)
    valid = cols < big
    rowg = r0 + lax.broadcasted_iota(jnp.int32, (br, K), 0)
    pos_ref[...] = jnp.where(valid, offs_i + jidx, trash)
    valc_ref[...] = jnp.where(valid, cols, 0).astype(jnp.float32)
    valr_ref[...] = jnp.where(valid, rowg, 0).astype(jnp.float32)

    off_smem[0] = base + jnp.sum(counts).astype(jnp.int32)


def _select(n1, n2, noise, trash):
    n = n1.shape[0]
    br = _pick_block_rows(n)
    grid = n // br
    body = functools.partial(_select_body, n=n, br=br, trash=trash)
    full = pl.BlockSpec((n, n1.shape[1]), lambda b: (0, 0))
    fullt = pl.BlockSpec((n1.shape[1], n), lambda b: (0, 0))
    blk = pl.BlockSpec((br, n), lambda b: (b, 0))
    out = pl.BlockSpec((br, K), lambda b: (b, 0))
    return pl.pallas_call(
        body,
        grid=(grid,),
        in_specs=[full, full, fullt, fullt, blk],
        out_specs=[out, out, out],
        out_shape=[
            jax.ShapeDtypeStruct((n, K), jnp.int32),
            jax.ShapeDtypeStruct((n, K), jnp.float32),
            jax.ShapeDtypeStruct((n, K), jnp.float32),
        ],
        scratch_shapes=[pltpu.SMEM((1,), jnp.int32)],
    )(n1, n2, n1.T, n2.T, noise)


def _scatter_body(pos_hbm, vc_hbm, vr_hbm, outr_hbm, outc_hbm,
                  idx_v, vc_v, vr_v, sem, *, chunks):
    wid = lax.axis_index("s") * 2 + lax.axis_index("c")
    pltpu.sync_copy(pos_hbm.at[wid], idx_v)
    pltpu.sync_copy(vc_hbm.at[wid], vc_v)
    pltpu.sync_copy(vr_hbm.at[wid], vr_v)

    def body(j, carry):
        pltpu.async_copy(vr_v.at[j], outr_hbm.at[idx_v.at[j]], sem).wait()
        pltpu.async_copy(vc_v.at[j], outc_hbm.at[idx_v.at[j]], sem).wait()
        return carry

    lax.fori_loop(0, chunks, body, 0)


def _worker_layout(x, n, rpw, chunks, pad_val):
    nk = x.shape[0] * K
    flat = x.reshape(-1)
    flat = jnp.pad(flat, (0, _NUM_WORKERS * rpw * K - nk),
                   constant_values=pad_val)
    per = flat.reshape(_NUM_WORKERS, rpw * K)
    per = jnp.pad(per, ((0, 0), (0, chunks * _LANES - rpw * K)),
                  constant_values=pad_val)
    return per.reshape(_NUM_WORKERS, chunks, _LANES)


def _scatter(pos, valc, valr, n, rpw, chunks, m_out):
    mesh = plsc.VectorSubcoreMesh(core_axis_name="c", subcore_axis_name="s")
    out = jax.ShapeDtypeStruct((m_out,), jnp.float32)
    kern = pl.kernel(
        functools.partial(_scatter_body, chunks=chunks),
        out_type=(out, out),
        mesh=mesh,
        scratch_types=[
            pltpu.VMEM((chunks, _LANES), jnp.int32),
            pltpu.VMEM((chunks, _LANES), jnp.float32),
            pltpu.VMEM((chunks, _LANES), jnp.float32),
            pltpu.SemaphoreType.DMA,
        ],
    )
    return kern(pos, valc, valr)


def kernel(emb1, emb2, W1, b1, W2, b2):
    n = emb1.shape[0]
    rpw = -(-n // _NUM_WORKERS)
    chunks = -(-(rpw * K) // _LANES)
    m_out = _NUM_WORKERS * chunks * _LANES
    trash = m_out - 8

    noise = jax.random.uniform(jax.random.key(1), (n, n),
                               dtype=jnp.float32) * 0.01
    n1, n2 = _nodevecs(emb1, emb2, W1, b1, W2, b2)
    pos, valc, valr = _select(n1, n2, noise, trash)

    posw = _worker_layout(pos, n, rpw, chunks, trash)
    vcw = _worker_layout(valc, n, rpw, chunks, 0.0)
    vrw = _worker_layout(valr, n, rpw, chunks, 0.0)
    outr, outc = _scatter(posw, vcw, vrw, n, rpw, chunks, m_out)
    return jnp.stack([outr[: n * K], outc[: n * K]])


# speedup vs baseline: 5.3847x; 1.1173x over previous
import functools

import jax
import jax.numpy as jnp
from jax import lax
from jax.experimental import pallas as pl
from jax.experimental.pallas import tpu as pltpu
from jax.experimental.pallas import tpu_sc as plsc

ALPHA = 3.0
K = 20

_NUM_WORKERS = 32
_LANES = 128


def _pick_block_rows(n):
    for br in (200, 128, 100, 64, 40, 16, 8):
        if n % br == 0 and br % 8 == 0:
            return br
    return n


def _nodevec_body(e1, w1, b1, e2, w2, b2, n1, n2):
    n1[...] = jnp.tanh(
        ALPHA * (lax.dot_general(e1[...], w1[...], (((1,), (1,)), ((), ())),
                                 preferred_element_type=jnp.float32) + b1[...]))
    n2[...] = jnp.tanh(
        ALPHA * (lax.dot_general(e2[...], w2[...], (((1,), (1,)), ((), ())),
                                 preferred_element_type=jnp.float32) + b2[...]))


def _nodevecs(emb1, emb2, W1, b1, W2, b2):
    n, d = emb1.shape
    out = jax.ShapeDtypeStruct((n, d), jnp.float32)
    return pl.pallas_call(
        _nodevec_body,
        out_shape=(out, out),
    )(emb1, W1, b1.reshape(1, d), emb2, W2, b2.reshape(1, d))


def _select_body(n1_ref, n2_ref, n1t_ref, n2t_ref, noise_ref,
                 pos_ref, valc_ref, valr_ref, off_smem, cstar_ref,
                 *, n, br, trash):
    b = pl.program_id(0)

    @pl.when(b == 0)
    def _():
        off_smem[0] = 0

    base = off_smem[0]
    r0 = b * br
    n1b = n1_ref[pl.ds(r0, br), :]
    n2b = n2_ref[pl.ds(r0, br), :]
    m1 = lax.dot_general(n1b, n2t_ref[...], (((1,), (0,)), ((), ())),
                         preferred_element_type=jnp.float32)
    m2 = lax.dot_general(n2b, n1t_ref[...], (((1,), (0,)), ((), ())),
                         preferred_element_type=jnp.float32)
    adj = jnp.maximum(jnp.tanh(ALPHA * (m1 - m2)), 0.0)
    v = adj + noise_ref[...]

    nfull = (n // 128) * 128
    fold = v[:, 0:128]
    for s in range(1, n // 128):
        fold = jnp.maximum(fold, v[:, s * 128:(s + 1) * 128])
    if nfull < n:
        lane = lax.broadcasted_iota(jnp.int32, (br, 128), 1)
        tail = jnp.where(lane >= 128 - (n - nfull), v[:, n - 128:n], 0.0)
        fold = jnp.maximum(fold, tail)
    g = fold
    s20 = jnp.max(g, axis=1, keepdims=True)
    for _ in range(K - 1):
        g = jnp.where(g == s20, -1.0, g)
        s20 = jnp.max(g, axis=1, keepdims=True)
    s20 = jnp.maximum(s20, 0.0)

    rowmax = jnp.max(v, axis=1, keepdims=True)
    hi = lax.bitcast_convert_type(rowmax, jnp.int32) + 1
    lo = lax.bitcast_convert_type(s20, jnp.int32)

    def bs_cond(carry):
        blo, bhi = carry
        return jnp.max(bhi - blo) > 1

    def bs_body(carry):
        blo, bhi = carry
        mid = (blo + bhi) // 2
        t = lax.bitcast_convert_type(mid, jnp.float32)
        cnt = jnp.sum((v >= t).astype(jnp.float32), axis=1, keepdims=True)
        ge = cnt >= float(K)
        return jnp.where(ge, mid, blo), jnp.where(ge, bhi, mid)

    lo, hi = lax.while_loop(bs_cond, bs_body, (lo, hi))
    t20 = lax.bitcast_convert_type(lo, jnp.float32)

    n_ge = jnp.sum((v >= t20).astype(jnp.float32), axis=1, keepdims=True)
    n_gt = jnp.sum((v > t20).astype(jnp.float32), axis=1, keepdims=True)
    r_need = float(K) - n_gt
    eq = v == t20
    colid = lax.broadcasted_iota(jnp.int32, (br, n), 1)
    cstar_ref[...] = jnp.full((br, 1), n - 1, jnp.int32)

    @pl.when(jnp.max(n_ge) > float(K))
    def _():
        rn = jnp.maximum(r_need, 1.0)
        clo = jnp.full((br, 1), -1, jnp.int32)
        chi = jnp.full((br, 1), n - 1, jnp.int32)

        def ts_body(_, carry):
            tlo, thi = carry
            mid = (tlo + thi) // 2
            cm = jnp.sum(jnp.where(eq & (colid <= mid), 1.0, 0.0), axis=1,
                         keepdims=True)
            ge = cm >= rn
            return jnp.where(ge, tlo, mid), jnp.where(ge, mid, thi)

        _, cs = lax.fori_loop(0, 15, ts_body, (clo, chi))
        cstar_ref[...] = cs

    cstar = cstar_ref[...]
    sel = (v > t20) | (eq & (colid <= cstar) & (r_need >= 1.0))
    kept = sel & (adj > 0.0)

    counts = jnp.sum(kept.astype(jnp.float32), axis=1, keepdims=True)
    tri = (lax.broadcasted_iota(jnp.int32, (br, br), 0)
           > lax.broadcasted_iota(jnp.int32, (br, br), 1)).astype(jnp.float32)
    offs = lax.dot_general(tri, counts, (((1,), (0,)), ((), ())),
                           preferred_element_type=jnp.float32)
    offs_i = base + offs.astype(jnp.int32)

    big = jnp.int32(n)
    rem = kept
    cols_list = []
    for _ in range(K):
        cj = jnp.min(jnp.where(rem, colid, big), axis=1, keepdims=True)
        cols_list.append(cj)
        rem = rem & (colid != cj)
    cols = jnp.concatenate(cols_list, axis=1)

    jidx = lax.broadcasted_iota(jnp.int32, (br, K), 1)
    valid = cols < big
    rowg = r0 + lax.broadcasted_iota(jnp.int32, (br, K), 0)
    pos_ref[...] = jnp.where(valid, offs_i + jidx, trash)
    valc_ref[...] = jnp.where(valid, cols, 0).astype(jnp.float32)
    valr_ref[...] = jnp.where(valid, rowg, 0).astype(jnp.float32)

    off_smem[0] = base + jnp.sum(counts).astype(jnp.int32)


def _select(n1, n2, noise, trash):
    n = n1.shape[0]
    br = _pick_block_rows(n)
    grid = n // br
    body = functools.partial(_select_body, n=n, br=br, trash=trash)
    full = pl.BlockSpec((n, n1.shape[1]), lambda b: (0, 0))
    fullt = pl.BlockSpec((n1.shape[1], n), lambda b: (0, 0))
    blk = pl.BlockSpec((br, n), lambda b: (b, 0))
    out = pl.BlockSpec((br, K), lambda b: (b, 0))
    return pl.pallas_call(
        body,
        grid=(grid,),
        in_specs=[full, full, fullt, fullt, blk],
        out_specs=[out, out, out],
        out_shape=[
            jax.ShapeDtypeStruct((n, K), jnp.int32),
            jax.ShapeDtypeStruct((n, K), jnp.float32),
            jax.ShapeDtypeStruct((n, K), jnp.float32),
        ],
        scratch_shapes=[pltpu.SMEM((1,), jnp.int32),
                        pltpu.VMEM((br, 1), jnp.int32)],
    )(n1, n2, n1.T, n2.T, noise)


def _scatter_body(pos_hbm, vc_hbm, vr_hbm, outr_hbm, outc_hbm,
                  idx_v, vc_v, vr_v, sem, *, chunks):
    wid = lax.axis_index("s") * 2 + lax.axis_index("c")
    pltpu.sync_copy(pos_hbm.at[wid], idx_v)
    pltpu.sync_copy(vc_hbm.at[wid], vc_v)
    pltpu.sync_copy(vr_hbm.at[wid], vr_v)

    def body(j, carry):
        pltpu.async_copy(vr_v.at[j], outr_hbm.at[idx_v.at[j]], sem).wait()
        pltpu.async_copy(vc_v.at[j], outc_hbm.at[idx_v.at[j]], sem).wait()
        return carry

    lax.fori_loop(0, chunks, body, 0)


def _worker_layout(x, n, rpw, chunks, pad_val):
    nk = x.shape[0] * K
    flat = x.reshape(-1)
    flat = jnp.pad(flat, (0, _NUM_WORKERS * rpw * K - nk),
                   constant_values=pad_val)
    per = flat.reshape(_NUM_WORKERS, rpw * K)
    per = jnp.pad(per, ((0, 0), (0, chunks * _LANES - rpw * K)),
                  constant_values=pad_val)
    return per.reshape(_NUM_WORKERS, chunks, _LANES)


def _scatter(pos, valc, valr, n, rpw, chunks, m_out):
    mesh = plsc.VectorSubcoreMesh(core_axis_name="c", subcore_axis_name="s")
    out = jax.ShapeDtypeStruct((m_out,), jnp.float32)
    kern = pl.kernel(
        functools.partial(_scatter_body, chunks=chunks),
        out_type=(out, out),
        mesh=mesh,
        scratch_types=[
            pltpu.VMEM((chunks, _LANES), jnp.int32),
            pltpu.VMEM((chunks, _LANES), jnp.float32),
            pltpu.VMEM((chunks, _LANES), jnp.float32),
            pltpu.SemaphoreType.DMA,
        ],
    )
    return kern(pos, valc, valr)


def kernel(emb1, emb2, W1, b1, W2, b2):
    n = emb1.shape[0]
    rpw = -(-n // _NUM_WORKERS)
    chunks = -(-(rpw * K) // _LANES)
    m_out = _NUM_WORKERS * chunks * _LANES
    trash = m_out - 8

    noise = jax.random.uniform(jax.random.key(1), (n, n),
                               dtype=jnp.float32) * 0.01
    n1, n2 = _nodevecs(emb1, emb2, W1, b1, W2, b2)
    pos, valc, valr = _select(n1, n2, noise, trash)

    posw = _worker_layout(pos, n, rpw, chunks, trash)
    vcw = _worker_layout(valc, n, rpw, chunks, 0.0)
    vrw = _worker_layout(valr, n, rpw, chunks, 0.0)
    outr, outc = _scatter(posw, vcw, vrw, n, rpw, chunks, m_out)
    return jnp.stack([outr[: n * K], outc[: n * K]])


# speedup vs baseline: 5.9734x; 1.1093x over previous
import functools

import jax
import jax.numpy as jnp
from jax import lax
from jax.experimental import pallas as pl
from jax.experimental.pallas import tpu as pltpu
from jax.experimental.pallas import tpu_sc as plsc

ALPHA = 3.0
K = 20

_NUM_WORKERS = 32
_LANES = 128


def _pick_block_rows(n):
    for br in (80, 200, 128, 100, 64, 40, 16, 8):
        if n % br == 0 and br % 8 == 0:
            return br
    return n


def _nodevec_body(e1, w1, b1, e2, w2, b2, n1, n2):
    n1[...] = jnp.tanh(
        ALPHA * (lax.dot_general(e1[...], w1[...], (((1,), (1,)), ((), ())),
                                 preferred_element_type=jnp.float32) + b1[...]))
    n2[...] = jnp.tanh(
        ALPHA * (lax.dot_general(e2[...], w2[...], (((1,), (1,)), ((), ())),
                                 preferred_element_type=jnp.float32) + b2[...]))


def _nodevecs(emb1, emb2, W1, b1, W2, b2):
    n, d = emb1.shape
    out = jax.ShapeDtypeStruct((n, d), jnp.float32)
    return pl.pallas_call(
        _nodevec_body,
        out_shape=(out, out),
    )(emb1, W1, b1.reshape(1, d), emb2, W2, b2.reshape(1, d))


def _select_body(n1_ref, n2_ref, n1t_ref, n2t_ref, noise_ref,
                 pos_ref, valc_ref, valr_ref, off_smem, cstar_ref,
                 *, n, br, trash):
    b = pl.program_id(0)

    @pl.when(b == 0)
    def _():
        off_smem[0] = 0

    base = off_smem[0]
    r0 = b * br
    n1b = n1_ref[pl.ds(r0, br), :]
    n2b = n2_ref[pl.ds(r0, br), :]
    m1 = lax.dot_general(n1b, n2t_ref[...], (((1,), (0,)), ((), ())),
                         preferred_element_type=jnp.float32)
    m2 = lax.dot_general(n2b, n1t_ref[...], (((1,), (0,)), ((), ())),
                         preferred_element_type=jnp.float32)
    adj = jnp.maximum(jnp.tanh(ALPHA * (m1 - m2)), 0.0)
    v = adj + noise_ref[...]

    nfull = (n // 128) * 128
    fold = v[:, 0:128]
    for s in range(1, n // 128):
        fold = jnp.maximum(fold, v[:, s * 128:(s + 1) * 128])
    if nfull < n:
        lane = lax.broadcasted_iota(jnp.int32, (br, 128), 1)
        tail = jnp.where(lane >= 128 - (n - nfull), v[:, n - 128:n], 0.0)
        fold = jnp.maximum(fold, tail)
    g = fold
    s20 = jnp.max(g, axis=1, keepdims=True)
    for _ in range(K - 1):
        g = jnp.where(g == s20, -1.0, g)
        s20 = jnp.max(g, axis=1, keepdims=True)
    s20 = jnp.maximum(s20, 0.0)

    rowmax = jnp.max(fold, axis=1, keepdims=True)
    hi = lax.bitcast_convert_type(rowmax, jnp.int32) + 1
    lo = lax.bitcast_convert_type(s20, jnp.int32)

    def bs_cond(carry):
        blo, bhi = carry
        return jnp.max(bhi - blo) > 1

    def bs_body(carry):
        blo, bhi = carry
        mid = (blo + bhi) // 2
        t = lax.bitcast_convert_type(mid, jnp.float32)
        cnt = jnp.sum((v >= t).astype(jnp.float32), axis=1, keepdims=True)
        ge = cnt >= float(K)
        return jnp.where(ge, mid, blo), jnp.where(ge, bhi, mid)

    lo, hi = lax.while_loop(bs_cond, bs_body, (lo, hi))
    t20 = lax.bitcast_convert_type(lo, jnp.float32)

    n_ge = jnp.sum((v >= t20).astype(jnp.float32), axis=1, keepdims=True)
    n_gt = jnp.sum((v > t20).astype(jnp.float32), axis=1, keepdims=True)
    r_need = float(K) - n_gt
    eq = v == t20
    colid = lax.broadcasted_iota(jnp.int32, (br, n), 1)
    cstar_ref[...] = jnp.full((br, 1), n - 1, jnp.int32)

    @pl.when(jnp.max(n_ge) > float(K))
    def _():
        rn = jnp.maximum(r_need, 1.0)
        clo = jnp.full((br, 1), -1, jnp.int32)
        chi = jnp.full((br, 1), n - 1, jnp.int32)

        def ts_body(_, carry):
            tlo, thi = carry
            mid = (tlo + thi) // 2
            cm = jnp.sum(jnp.where(eq & (colid <= mid), 1.0, 0.0), axis=1,
                         keepdims=True)
            ge = cm >= rn
            return jnp.where(ge, tlo, mid), jnp.where(ge, mid, thi)

        _, cs = lax.fori_loop(0, 15, ts_body, (clo, chi))
        cstar_ref[...] = cs

    cstar = cstar_ref[...]
    sel = (v > t20) | (eq & (colid <= cstar) & (r_need >= 1.0))
    kept = sel & (adj > 0.0)

    counts = jnp.sum(kept.astype(jnp.float32), axis=1, keepdims=True)
    tri = (lax.broadcasted_iota(jnp.int32, (br, br), 0)
           > lax.broadcasted_iota(jnp.int32, (br, br), 1)).astype(jnp.float32)
    offs = lax.dot_general(tri, counts, (((1,), (0,)), ((), ())),
                           preferred_element_type=jnp.float32)
    offs_i = base + offs.astype(jnp.int32)

    big = jnp.int32(n)
    w = jnp.where(kept, colid, big)
    cols_list = []
    for _ in range(K):
        cj = jnp.min(w, axis=1, keepdims=True)
        cols_list.append(cj)
        w = jnp.where(w == cj, big, w)
    cols = jnp.concatenate(cols_list, axis=1)

    jidx = lax.broadcasted_iota(jnp.int32, (br, K), 1)
    valid = cols < big
    rowg = r0 + lax.broadcasted_iota(jnp.int32, (br, K), 0)
    pos_ref[...] = jnp.where(valid, offs_i + jidx, trash)
    valc_ref[...] = jnp.where(valid, cols, 0).astype(jnp.float32)
    valr_ref[...] = jnp.where(valid, rowg, 0).astype(jnp.float32)

    off_smem[0] = base + jnp.sum(counts).astype(jnp.int32)


def _select(n1, n2, noise, trash):
    n = n1.shape[0]
    br = _pick_block_rows(n)
    grid = n // br
    body = functools.partial(_select_body, n=n, br=br, trash=trash)
    full = pl.BlockSpec((n, n1.shape[1]), lambda b: (0, 0))
    fullt = pl.BlockSpec((n1.shape[1], n), lambda b: (0, 0))
    blk = pl.BlockSpec((br, n), lambda b: (b, 0))
    out = pl.BlockSpec((br, K), lambda b: (b, 0))
    return pl.pallas_call(
        body,
        grid=(grid,),
        in_specs=[full, full, fullt, fullt, blk],
        out_specs=[out, out, out],
        out_shape=[
            jax.ShapeDtypeStruct((n, K), jnp.int32),
            jax.ShapeDtypeStruct((n, K), jnp.float32),
            jax.ShapeDtypeStruct((n, K), jnp.float32),
        ],
        scratch_shapes=[pltpu.SMEM((1,), jnp.int32),
                        pltpu.VMEM((br, 1), jnp.int32)],
    )(n1, n2, n1.T, n2.T, noise)


def _scatter_body(pos_hbm, vc_hbm, vr_hbm, outr_hbm, outc_hbm,
                  idx_v, vc_v, vr_v, sem, *, chunks):
    wid = lax.axis_index("s") * 2 + lax.axis_index("c")
    pltpu.sync_copy(pos_hbm.at[wid], idx_v)
    pltpu.sync_copy(vc_hbm.at[wid], vc_v)
    pltpu.sync_copy(vr_hbm.at[wid], vr_v)

    def body(j, carry):
        pltpu.async_copy(vr_v.at[j], outr_hbm.at[idx_v.at[j]], sem).wait()
        pltpu.async_copy(vc_v.at[j], outc_hbm.at[idx_v.at[j]], sem).wait()
        return carry

    lax.fori_loop(0, chunks, body, 0)


def _worker_layout(x, n, rpw, chunks, pad_val):
    nk = x.shape[0] * K
    flat = x.reshape(-1)
    flat = jnp.pad(flat, (0, _NUM_WORKERS * rpw * K - nk),
                   constant_values=pad_val)
    per = flat.reshape(_NUM_WORKERS, rpw * K)
    per = jnp.pad(per, ((0, 0), (0, chunks * _LANES - rpw * K)),
                  constant_values=pad_val)
    return per.reshape(_NUM_WORKERS, chunks, _LANES)


def _scatter(pos, valc, valr, n, rpw, chunks, m_out):
    mesh = plsc.VectorSubcoreMesh(core_axis_name="c", subcore_axis_name="s")
    out = jax.ShapeDtypeStruct((m_out,), jnp.float32)
    kern = pl.kernel(
        functools.partial(_scatter_body, chunks=chunks),
        out_type=(out, out),
        mesh=mesh,
        scratch_types=[
            pltpu.VMEM((chunks, _LANES), jnp.int32),
            pltpu.VMEM((chunks, _LANES), jnp.float32),
            pltpu.VMEM((chunks, _LANES), jnp.float32),
            pltpu.SemaphoreType.DMA,
        ],
    )
    return kern(pos, valc, valr)


def kernel(emb1, emb2, W1, b1, W2, b2):
    n = emb1.shape[0]
    rpw = -(-n // _NUM_WORKERS)
    chunks = -(-(rpw * K) // _LANES)
    m_out = _NUM_WORKERS * chunks * _LANES
    trash = m_out - 8

    noise = jax.random.uniform(jax.random.key(1), (n, n),
                               dtype=jnp.float32) * 0.01
    n1, n2 = _nodevecs(emb1, emb2, W1, b1, W2, b2)
    pos, valc, valr = _select(n1, n2, noise, trash)

    posw = _worker_layout(pos, n, rpw, chunks, trash)
    vcw = _worker_layout(valc, n, rpw, chunks, 0.0)
    vrw = _worker_layout(valr, n, rpw, chunks, 0.0)
    outr, outc = _scatter(posw, vcw, vrw, n, rpw, chunks, m_out)
    return jnp.stack([outr[: n * K], outc[: n * K]])


# speedup vs baseline: 5.9759x; 1.0004x over previous
import functools

import jax
import jax.numpy as jnp
from jax import lax
from jax.experimental import pallas as pl
from jax.experimental.pallas import tpu as pltpu
from jax.experimental.pallas import tpu_sc as plsc

ALPHA = 3.0
K = 20

_NUM_WORKERS = 32
_LANES = 128


def _pick_block_rows(n):
    for br in (80, 200, 128, 100, 64, 40, 16, 8):
        if n % br == 0 and br % 8 == 0:
            return br
    return n


def _nodevec_body(e1, w1, b1, e2, w2, b2, n1, n2):
    n1[...] = jnp.tanh(
        ALPHA * (lax.dot_general(e1[...], w1[...], (((1,), (1,)), ((), ())),
                                 preferred_element_type=jnp.float32) + b1[...]))
    n2[...] = jnp.tanh(
        ALPHA * (lax.dot_general(e2[...], w2[...], (((1,), (1,)), ((), ())),
                                 preferred_element_type=jnp.float32) + b2[...]))


def _nodevecs(emb1, emb2, W1, b1, W2, b2):
    n, d = emb1.shape
    out = jax.ShapeDtypeStruct((n, d), jnp.float32)
    return pl.pallas_call(
        _nodevec_body,
        out_shape=(out, out),
    )(emb1, W1, b1.reshape(1, d), emb2, W2, b2.reshape(1, d))


def _select_body(n1_ref, n2_ref, n1t_ref, n2t_ref, noise_ref,
                 pos_ref, valc_ref, valr_ref, off_smem, cstar_ref,
                 *, n, br, trash):
    b = pl.program_id(0)

    @pl.when(b == 0)
    def _():
        off_smem[0] = 0

    base = off_smem[0]
    r0 = b * br
    n1b = n1_ref[pl.ds(r0, br), :]
    n2b = n2_ref[pl.ds(r0, br), :]
    m1 = lax.dot_general(n1b, n2t_ref[...], (((1,), (0,)), ((), ())),
                         preferred_element_type=jnp.float32)
    m2 = lax.dot_general(n2b, n1t_ref[...], (((1,), (0,)), ((), ())),
                         preferred_element_type=jnp.float32)
    adj = jnp.maximum(jnp.tanh(ALPHA * (m1 - m2)), 0.0)
    v = adj + noise_ref[...]

    nfull = (n // 128) * 128
    fold = v[:, 0:128]
    for s in range(1, n // 128):
        fold = jnp.maximum(fold, v[:, s * 128:(s + 1) * 128])
    if nfull < n:
        lane = lax.broadcasted_iota(jnp.int32, (br, 128), 1)
        tail = jnp.where(lane >= 128 - (n - nfull), v[:, n - 128:n], 0.0)
        fold = jnp.maximum(fold, tail)
    g = fold
    s20 = jnp.max(g, axis=1, keepdims=True)
    for _ in range(K - 1):
        g = jnp.where(g == s20, -1.0, g)
        s20 = jnp.max(g, axis=1, keepdims=True)
    s20 = jnp.maximum(s20, 0.0)

    rowmax = jnp.max(fold, axis=1, keepdims=True)
    hi = lax.bitcast_convert_type(rowmax, jnp.int32) + 1
    lo = lax.bitcast_convert_type(s20, jnp.int32)

    def bs_cond(carry):
        blo, bhi = carry
        return jnp.max(bhi - blo) > 1

    def bs_body(carry):
        blo, bhi = carry
        mid = (blo + bhi) // 2
        t = lax.bitcast_convert_type(mid, jnp.float32)
        cnt = jnp.sum((v >= t).astype(jnp.float32), axis=1, keepdims=True)
        ge = cnt >= float(K)
        return jnp.where(ge, mid, blo), jnp.where(ge, bhi, mid)

    lo, hi = lax.while_loop(bs_cond, bs_body, (lo, hi))
    t20 = lax.bitcast_convert_type(lo, jnp.float32)

    n_ge = jnp.sum((v >= t20).astype(jnp.float32), axis=1, keepdims=True)
    n_gt = jnp.sum((v > t20).astype(jnp.float32), axis=1, keepdims=True)
    r_need = float(K) - n_gt
    eq = v == t20
    colid = lax.broadcasted_iota(jnp.int32, (br, n), 1)
    cstar_ref[...] = jnp.full((br, 1), n - 1, jnp.int32)

    @pl.when(jnp.max(n_ge) > float(K))
    def _():
        rn = jnp.maximum(r_need, 1.0)
        clo = jnp.full((br, 1), -1, jnp.int32)
        chi = jnp.full((br, 1), n - 1, jnp.int32)

        def ts_body(_, carry):
            tlo, thi = carry
            mid = (tlo + thi) // 2
            cm = jnp.sum(jnp.where(eq & (colid <= mid), 1.0, 0.0), axis=1,
                         keepdims=True)
            ge = cm >= rn
            return jnp.where(ge, tlo, mid), jnp.where(ge, mid, thi)

        _, cs = lax.fori_loop(0, 15, ts_body, (clo, chi))
        cstar_ref[...] = cs

    cstar = cstar_ref[...]
    sel = (v > t20) | (eq & (colid <= cstar) & (r_need >= 1.0))
    kept = sel & (adj > 0.0)

    counts = jnp.sum(kept.astype(jnp.float32), axis=1, keepdims=True)
    tri = (lax.broadcasted_iota(jnp.int32, (br, br), 0)
           > lax.broadcasted_iota(jnp.int32, (br, br), 1)).astype(jnp.float32)
    offs = lax.dot_general(tri, counts, (((1,), (0,)), ((), ())),
                           preferred_element_type=jnp.float32)
    offs_i = base + offs.astype(jnp.int32)

    big = jnp.int32(n)
    w = jnp.where(kept, colid, big)
    cols_list = []
    for _ in range(K):
        cj = jnp.min(w, axis=1, keepdims=True)
        cols_list.append(cj)
        w = jnp.where(w == cj, big, w)
    cols = jnp.concatenate(cols_list, axis=1)

    jidx = lax.broadcasted_iota(jnp.int32, (br, K), 1)
    valid = cols < big
    rowg = r0 + lax.broadcasted_iota(jnp.int32, (br, K), 0)
    pos_ref[...] = jnp.where(valid, offs_i + jidx, trash)
    valc_ref[...] = jnp.where(valid, cols, 0).astype(jnp.float32)
    valr_ref[...] = jnp.where(valid, rowg, 0).astype(jnp.float32)

    off_smem[0] = base + jnp.sum(counts).astype(jnp.int32)


def _select(n1, n2, noise, trash):
    n = n1.shape[0]
    br = _pick_block_rows(n)
    grid = n // br
    body = functools.partial(_select_body, n=n, br=br, trash=trash)
    full = pl.BlockSpec((n, n1.shape[1]), lambda b: (0, 0))
    fullt = pl.BlockSpec((n1.shape[1], n), lambda b: (0, 0))
    blk = pl.BlockSpec((br, n), lambda b: (b, 0))
    out = pl.BlockSpec((br, K), lambda b: (b, 0))
    return pl.pallas_call(
        body,
        grid=(grid,),
        in_specs=[full, full, fullt, fullt, blk],
        out_specs=[out, out, out],
        out_shape=[
            jax.ShapeDtypeStruct((n, K), jnp.int32),
            jax.ShapeDtypeStruct((n, K), jnp.float32),
            jax.ShapeDtypeStruct((n, K), jnp.float32),
        ],
        scratch_shapes=[pltpu.SMEM((1,), jnp.int32),
                        pltpu.VMEM((br, 1), jnp.int32)],
    )(n1, n2, n1.T, n2.T, noise)


def _scatter_body(pos_hbm, vc_hbm, vr_hbm, outr_hbm, outc_hbm,
                  idx_v, vc_v, vr_v, sem, *, chunks):
    wid = lax.axis_index("s") * 2 + lax.axis_index("c")
    pltpu.sync_copy(pos_hbm.at[wid], idx_v)
    pltpu.sync_copy(vc_hbm.at[wid], vc_v)
    pltpu.sync_copy(vr_hbm.at[wid], vr_v)

    def body(j, carry):
        cr = pltpu.async_copy(vr_v.at[j], outr_hbm.at[idx_v.at[j]], sem)
        cc = pltpu.async_copy(vc_v.at[j], outc_hbm.at[idx_v.at[j]], sem)
        cr.wait()
        cc.wait()
        return carry

    lax.fori_loop(0, chunks, body, 0)


def _worker_layout(x, n, rpw, chunks, pad_val):
    nk = x.shape[0] * K
    flat = x.reshape(-1)
    flat = jnp.pad(flat, (0, _NUM_WORKERS * rpw * K - nk),
                   constant_values=pad_val)
    per = flat.reshape(_NUM_WORKERS, rpw * K)
    per = jnp.pad(per, ((0, 0), (0, chunks * _LANES - rpw * K)),
                  constant_values=pad_val)
    return per.reshape(_NUM_WORKERS, chunks, _LANES)


def _scatter(pos, valc, valr, n, rpw, chunks, m_out):
    mesh = plsc.VectorSubcoreMesh(core_axis_name="c", subcore_axis_name="s")
    out = jax.ShapeDtypeStruct((m_out,), jnp.float32)
    kern = pl.kernel(
        functools.partial(_scatter_body, chunks=chunks),
        out_type=(out, out),
        mesh=mesh,
        scratch_types=[
            pltpu.VMEM((chunks, _LANES), jnp.int32),
            pltpu.VMEM((chunks, _LANES), jnp.float32),
            pltpu.VMEM((chunks, _LANES), jnp.float32),
            pltpu.SemaphoreType.DMA,
        ],
    )
    return kern(pos, valc, valr)


def kernel(emb1, emb2, W1, b1, W2, b2):
    n = emb1.shape[0]
    rpw = -(-n // _NUM_WORKERS)
    chunks = -(-(rpw * K) // _LANES)
    m_out = _NUM_WORKERS * chunks * _LANES
    trash = m_out - 8

    noise = jax.random.uniform(jax.random.key(1), (n, n),
                               dtype=jnp.float32) * 0.01
    n1, n2 = _nodevecs(emb1, emb2, W1, b1, W2, b2)
    pos, valc, valr = _select(n1, n2, noise, trash)

    posw = _worker_layout(pos, n, rpw, chunks, trash)
    vcw = _worker_layout(valc, n, rpw, chunks, 0.0)
    vrw = _worker_layout(valr, n, rpw, chunks, 0.0)
    outr, outc = _scatter(posw, vcw, vrw, n, rpw, chunks, m_out)
    return jnp.stack([outr[: n * K], outc[: n * K]])
